# baseline (speedup 1.0000x reference)
import functools

import jax
import jax.numpy as jnp
from jax import lax
from jax.experimental import pallas as pl
from jax.experimental.pallas import tpu as pltpu
from jax.experimental.pallas import tpu_sc as plsc

VOCAB = 100000
VEC_DIM = 128
HIDDEN = 256
LABELS = 16
BATCH = 4096
SEQ = 200

NC = 2
NS = 16
NW = NC * NS
HALF = SEQ // 2
RING = 6

BATCH_A = 1280
BATCH_B = BATCH - BATCH_A

PACK_BLK = 5000


def _pack_words(e):
  bf = e.astype(jnp.bfloat16)
  bits = lax.bitcast_convert_type(bf, jnp.uint16).astype(jnp.int32)
  return bits[:, :64] | (bits[:, 64:] << 16)


def _pack_kernel(e1_ref, e2_ref, out_ref):
  out_ref[...] = jnp.concatenate(
      [_pack_words(e1_ref[...]), _pack_words(e2_ref[...])], axis=1)


def _pack_table(embed):
  hblk = (VOCAB // 2) // PACK_BLK
  return pl.pallas_call(
      _pack_kernel,
      grid=(hblk,),
      in_specs=[
          pl.BlockSpec((PACK_BLK, VEC_DIM), lambda i: (i, 0)),
          pl.BlockSpec((PACK_BLK, VEC_DIM), lambda i, h=hblk: (i + h, 0)),
      ],
      out_specs=pl.BlockSpec((PACK_BLK, VEC_DIM), lambda i: (i, 0)),
      out_shape=jax.ShapeDtypeStruct((VOCAB // 2, VEC_DIM), jnp.int32),
  )(embed, embed)


def _make_gather_sum(n_batch, bf16_mode):
  n_elem_w = n_batch // NW
  rows_w = 2 * n_elem_w
  n_full = rows_w // RING
  rem = rows_w % RING
  assert rem % 2 == 0
  lanes = VEC_DIM // 2 if bf16_mode else VEC_DIM
  dtype = jnp.int32 if bf16_mode else jnp.float32
  mesh = plsc.VectorSubcoreMesh(core_axis_name="c", subcore_axis_name="s")

  @functools.partial(
      pl.kernel,
      mesh=mesh,
      out_type=jax.ShapeDtypeStruct((n_batch, VEC_DIM), jnp.float32),
      compiler_params=(pltpu.CompilerParams(use_tc_tiling_on_sc=False)
                       if bf16_mode else None),
      scratch_types=[
          pltpu.VMEM((rows_w, HALF), jnp.int32),
      ] + [pltpu.VMEM((HALF, lanes), dtype) for _ in range(RING)] + [
          pltpu.VMEM((n_elem_w, VEC_DIM), jnp.float32),
      ] + [pltpu.SemaphoreType.DMA for _ in range(RING)],
  )
  def k(table_hbm, xr_hbm, out_hbm, idx_v, *rest):
    bufs = rest[:RING]
    m_local = rest[RING]
    sems = rest[RING + 1:]
    wid = lax.axis_index("s") * NC + lax.axis_index("c")
    pltpu.sync_copy(xr_hbm.at[pl.ds(wid * rows_w, rows_w)], idx_v)

    def gather_half(r, slot):
      pltpu.make_async_copy(
          table_hbm.at[idx_v.at[r]], bufs[slot], sems[slot]).start()

    def wait_half(slot):
      pltpu.make_async_copy(
          table_hbm.at[idx_v.at[0]], bufs[slot], sems[slot]).wait()

    if bf16_mode:
      def accum_row(buf, j, accs):
        for d in range(4):
          w = buf[j, d * 16:(d + 1) * 16]
          a = lax.bitcast_convert_type(lax.shift_left(w, 16), jnp.float32)
          b = lax.bitcast_convert_type(
              lax.bitwise_and(w, jnp.int32(-65536)), jnp.float32)
          accs[d] = accs[d] + a
          accs[d + 4] = accs[d + 4] + b
        return accs
    else:
      def accum_row(buf, j, accs):
        for d in range(8):
          accs[d] = accs[d] + buf[j, d * 16:(d + 1) * 16]
        return accs

    def reduce_half(buf, accs):
      def body(j4, accs):
        for u in range(4):
          accs = tuple(accum_row(buf, 4 * j4 + u, list(accs)))
        return accs
      return lax.fori_loop(0, HALF // 4, body, accs)

    def step(s, r0, base_b, accs):
      wait_half(s)
      if s % 2 == 0:
        accs = tuple(jnp.zeros((16,), jnp.float32) for _ in range(8))
      accs = reduce_half(bufs[s], accs)
      if s % 2 == 1:
        b = base_b + s // 2
        for d in range(8):
          m_local[b, d * 16:(d + 1) * 16] = accs[d]
      return accs

    for slot in range(RING - 1):
      gather_half(slot, slot)

    def outer(g, carry):
      r0 = RING * g
      accs = None
      for s in range(RING):
        @pl.when(r0 + s + RING - 1 < rows_w)
        def _(s=s):
          gather_half(r0 + s + RING - 1, (s + RING - 1) % RING)
        accs = step(s, r0, (RING // 2) * g, accs)
      return carry

    lax.fori_loop(0, n_full, outer, 0)
    accs = None
    for s in range(rem):
      accs = step(s, RING * n_full, (RING // 2) * n_full, accs)
    pltpu.sync_copy(m_local, out_hbm.at[pl.ds(wid * n_elem_w, n_elem_w)])

  return k


def _mlp_kernel(ma_ref, mb_ref, w1_ref, b1_ref, g_ref, be_ref, w2_ref,
                b2_ref, out_ref):
  m = jnp.concatenate([ma_ref[...], mb_ref[...]], axis=0) * (1.0 / SEQ)
  h = lax.dot_general(m, w1_ref[...], (((1,), (1,)), ((), ())),
                      preferred_element_type=jnp.float32)
  h = h + b1_ref[...]
  mu = jnp.mean(h, axis=0, keepdims=True)
  d = h - mu
  var = jnp.mean(d * d, axis=0, keepdims=True)
  hn = d * lax.rsqrt(var + 1e-5) * g_ref[...] + be_ref[...]
  a = jnp.maximum(hn, 0.0)
  out_ref[...] = lax.dot_general(a, w2_ref[...], (((1,), (1,)), ((), ())),
                                 preferred_element_type=jnp.float32) + b2_ref[...]


def kernel(X, embed, W1, b1, gamma, beta, W2, b2):
  xi = X.astype(jnp.int32)
  xr_a = xi[:BATCH_A].reshape(BATCH_A * 2, HALF)
  xb = xi[BATCH_A:]
  xr_b = jnp.where(xb < VOCAB // 2, 2 * xb,
                   2 * xb - (VOCAB - 1)).reshape(BATCH_B * 2, HALF)
  msum_a = _make_gather_sum(BATCH_A, False)(embed, xr_a)
  packed = _pack_table(embed).reshape(VOCAB, VEC_DIM // 2)
  msum_b = _make_gather_sum(BATCH_B, True)(packed, xr_b)
  out = pl.pallas_call(
      _mlp_kernel,
      out_shape=jax.ShapeDtypeStruct((BATCH, LABELS), jnp.float32),
  )(msum_a, msum_b, W1, b1.reshape(1, HIDDEN), gamma.reshape(1, HIDDEN),
    beta.reshape(1, HIDDEN), W2, b2.reshape(1, LABELS))
  return out


# speedup vs baseline: 1.1234x; 1.1234x over previous
import functools

import jax
import jax.numpy as jnp
from jax import lax
from jax.experimental import pallas as pl
from jax.experimental.pallas import tpu as pltpu
from jax.experimental.pallas import tpu_sc as plsc

VOCAB = 100000
VEC_DIM = 128
HIDDEN = 256
LABELS = 16
BATCH = 4096
SEQ = 200

NC = 2
NS = 16
NW = NC * NS
HALF = SEQ // 2
RING = 6

IDXW = 128

PACK_BLK = 5000


def _pack_words(e):
  bf = e.astype(jnp.bfloat16)
  bits = lax.bitcast_convert_type(bf, jnp.uint16).astype(jnp.int32)
  return bits[:, :64] | (bits[:, 64:] << 16)


def _pack_kernel(e1_ref, e2_ref, out_ref):
  out_ref[...] = jnp.concatenate(
      [_pack_words(e1_ref[...]), _pack_words(e2_ref[...])], axis=1)


def _pack_table(embed):
  hblk = (VOCAB // 2) // PACK_BLK
  return pl.pallas_call(
      _pack_kernel,
      grid=(hblk,),
      in_specs=[
          pl.BlockSpec((PACK_BLK, VEC_DIM), lambda i: (i, 0)),
          pl.BlockSpec((PACK_BLK, VEC_DIM), lambda i, h=hblk: (i + h, 0)),
      ],
      out_specs=pl.BlockSpec((PACK_BLK, VEC_DIM), lambda i: (i, 0)),
      out_shape=jax.ShapeDtypeStruct((VOCAB // 2, VEC_DIM), jnp.int32),
  )(embed, embed)


def _make_gather_sum(n_batch):
  n_elem_w = n_batch // NW
  rows_w = 2 * n_elem_w
  nring = 4
  assert n_elem_w % nring == 0
  lanes = VEC_DIM // 2
  mesh = plsc.VectorSubcoreMesh(core_axis_name="c", subcore_axis_name="s")

  @functools.partial(
      pl.kernel,
      mesh=mesh,
      out_type=jax.ShapeDtypeStruct((n_batch, VEC_DIM), jnp.float32),
      compiler_params=pltpu.CompilerParams(use_tc_tiling_on_sc=False),
      scratch_types=[
          pltpu.VMEM((rows_w, IDXW), jnp.int32),
      ] + [pltpu.VMEM((SEQ, lanes), jnp.int32) for _ in range(nring)] + [
          pltpu.VMEM((n_elem_w, VEC_DIM), jnp.float32),
      ] + [pltpu.SemaphoreType.DMA for _ in range(nring)],
  )
  def k(table_hbm, xr_hbm, out_hbm, idx_v, *rest):
    bufs = rest[:nring]
    m_local = rest[nring]
    sems = rest[nring + 1:]
    wid = lax.axis_index("s") * NC + lax.axis_index("c")
    pltpu.sync_copy(xr_hbm.at[pl.ds(wid * rows_w, rows_w)], idx_v)

    def gather_elem(b, slot):
      pltpu.make_async_copy(
          table_hbm.at[idx_v.at[2 * b, pl.ds(0, 96)]],
          bufs[slot].at[pl.ds(0, 96)], sems[slot]).start()
      pltpu.make_async_copy(
          table_hbm.at[idx_v.at[2 * b + 1, pl.ds(0, 104)]],
          bufs[slot].at[pl.ds(96, 104)], sems[slot]).start()

    def wait_elem(slot):
      pltpu.make_async_copy(
          table_hbm.at[idx_v.at[0, pl.ds(0, 96)]],
          bufs[slot].at[pl.ds(0, 96)], sems[slot]).wait()
      pltpu.make_async_copy(
          table_hbm.at[idx_v.at[0, pl.ds(0, 104)]],
          bufs[slot].at[pl.ds(96, 104)], sems[slot]).wait()

    def accum_row(buf, j, accs):
      for d in range(4):
        w = buf[j, d * 16:(d + 1) * 16]
        a = lax.bitcast_convert_type(lax.shift_left(w, 16), jnp.float32)
        b = lax.bitcast_convert_type(
            lax.bitwise_and(w, jnp.int32(-65536)), jnp.float32)
        accs[d] = accs[d] + a
        accs[d + 4] = accs[d + 4] + b
      return accs

    def reduce_elem(buf, b):
      def body(j4, accs):
        for u in range(4):
          accs = tuple(accum_row(buf, 4 * j4 + u, list(accs)))
        return accs
      accs = tuple(jnp.zeros((16,), jnp.float32) for _ in range(8))
      accs = lax.fori_loop(0, SEQ // 4, body, accs)
      for d in range(8):
        m_local[b, d * 16:(d + 1) * 16] = accs[d]

    for slot in range(nring - 1):
      gather_elem(slot, slot)

    def outer(g, carry):
      b0 = nring * g
      for s in range(nring):
        @pl.when(b0 + s + nring - 1 < n_elem_w)
        def _(s=s):
          gather_elem(b0 + s + nring - 1, (s + nring - 1) % nring)
        wait_elem(s)
        reduce_elem(bufs[s], b0 + s)
      return carry

    lax.fori_loop(0, n_elem_w // nring, outer, 0)
    pltpu.sync_copy(m_local, out_hbm.at[pl.ds(wid * n_elem_w, n_elem_w)])

  return k


def _mlp_kernel(msum_ref, w1_ref, b1_ref, g_ref, be_ref, w2_ref,
                b2_ref, out_ref):
  m = msum_ref[...] * (1.0 / SEQ)
  h = lax.dot_general(m, w1_ref[...], (((1,), (1,)), ((), ())),
                      preferred_element_type=jnp.float32)
  h = h + b1_ref[...]
  mu = jnp.mean(h, axis=0, keepdims=True)
  d = h - mu
  var = jnp.mean(d * d, axis=0, keepdims=True)
  hn = d * lax.rsqrt(var + 1e-5) * g_ref[...] + be_ref[...]
  a = jnp.maximum(hn, 0.0)
  out_ref[...] = lax.dot_general(a, w2_ref[...], (((1,), (1,)), ((), ())),
                                 preferred_element_type=jnp.float32) + b2_ref[...]


def kernel(X, embed, W1, b1, gamma, beta, W2, b2):
  xi = X.astype(jnp.int32)
  xrm = jnp.where(xi < VOCAB // 2, 2 * xi, 2 * xi - (VOCAB - 1))
  ra = jnp.pad(xrm[:, :96], ((0, 0), (0, IDXW - 96)))
  rb = jnp.pad(xrm[:, 96:], ((0, 0), (0, IDXW - 104)))
  xr = jnp.stack([ra, rb], axis=1).reshape(BATCH * 2, IDXW)
  packed = _pack_table(embed).reshape(VOCAB, VEC_DIM // 2)
  msum = _make_gather_sum(BATCH)(packed, xr)
  out = pl.pallas_call(
      _mlp_kernel,
      out_shape=jax.ShapeDtypeStruct((BATCH, LABELS), jnp.float32),
  )(msum, W1, b1.reshape(1, HIDDEN), gamma.reshape(1, HIDDEN),
    beta.reshape(1, HIDDEN), W2, b2.reshape(1, LABELS))
  return out
